# baseline (speedup 1.0000x reference)
import functools

import jax
import jax.numpy as jnp
from jax import lax
from jax.experimental import pallas as pl
from jax.experimental.pallas import tpu as pltpu
from jax.experimental.pallas import tpu_sc as plsc

_VOCAB = 100000
_D = 64
_B = 1024
_HIST = 50

_NC, _NS = 2, 16
_NW = _NC * _NS
_ROWS_PER_W = _B // _NW
_GSZ = 8
_NG = _ROWS_PER_W // _GSZ
_IPG = _GSZ * _HIST
_VPG = _IPG // 16
_LANES = 16
_mesh = plsc.VectorSubcoreMesh(core_axis_name="c", subcore_axis_name="s")


@functools.partial(
    pl.kernel,
    out_type=jax.ShapeDtypeStruct((_B, _D), jnp.float32),
    mesh=_mesh,
    scratch_types=[
        pltpu.VMEM((_ROWS_PER_W * _HIST,), jnp.int32),
        pltpu.VMEM((_NG, _IPG, _D // 2), jnp.int32),
        pltpu.VMEM((_ROWS_PER_W, _D), jnp.float32),
        [pltpu.SemaphoreType.DMA for _ in range(_NG)],
    ],
    compiler_params=pltpu.CompilerParams(use_tc_tiling_on_sc=False),
)
def _sc_pool(idx_hbm, table_hbm, x_hbm, idx_v, rows_v, xout_v, sems):
    wid = lax.axis_index("s") * _NC + lax.axis_index("c")
    base = wid * _ROWS_PER_W
    pltpu.sync_copy(idx_hbm.at[wid], idx_v)

    def issue_group(g):
        for u in range(_VPG):
            iv = idx_v[pl.ds(g * _IPG + u * _LANES, _LANES)]
            pltpu.async_copy(table_hbm.at[iv],
                             rows_v.at[g, pl.ds(u * _LANES, _LANES)], sems[g])

    issue_group(0)
    issue_group(1)

    for g in range(_NG):
        if g + 2 < _NG:
            issue_group(g + 2)
        for u in range(_VPG):
            iv = idx_v[pl.ds(g * _IPG + u * _LANES, _LANES)]
            pltpu.make_async_copy(
                table_hbm.at[iv],
                rows_v.at[g, pl.ds(u * _LANES, _LANES)], sems[g]).wait()
        for r in range(_GSZ):
            def acc_body(j, accs):
                out = []
                shift = jnp.full((_LANES,), 16, jnp.int32)
                mask = jnp.full((_LANES,), -65536, jnp.int32)
                for c in range(2):
                    w = rows_v[g, r * _HIST + j, pl.ds(_LANES * c, _LANES)]
                    lo = lax.bitcast_convert_type(
                        lax.shift_left(w, shift), jnp.float32)
                    hi = lax.bitcast_convert_type(
                        lax.bitwise_and(w, mask), jnp.float32)
                    out.append(accs[2 * c] + lo)
                    out.append(accs[2 * c + 1] + hi)
                return tuple(out)

            accs = lax.fori_loop(
                0, _HIST, acc_body,
                tuple(jnp.zeros((_LANES,), jnp.float32) for _ in range(4)))
            for k in range(4):
                xout_v[g * _GSZ + r, pl.ds(k * _LANES, _LANES)] = (
                    accs[k] * (1.0 / _HIST))

    pltpu.sync_copy(xout_v, x_hbm.at[pl.ds(base, _ROWS_PER_W)])


_VT = 2048


def _mm_body(w_ref, x_ref, b_ref, o_ref):
    o_ref[...] = lax.dot_general(
        w_ref[...], x_ref[...],
        dimension_numbers=(((1,), (1,)), ((), ())),
        preferred_element_type=jnp.float32,
    ) + b_ref[...]


def _project(x, W, bcol):
    out_t = pl.pallas_call(
        _mm_body,
        grid=(pl.cdiv(_VOCAB, _VT),),
        in_specs=[
            pl.BlockSpec((_VT, _D), lambda i: (i, 0)),
            pl.BlockSpec((_B, _D), lambda i: (0, 0)),
            pl.BlockSpec((_VT, 1), lambda i: (i, 0)),
        ],
        out_specs=pl.BlockSpec((_VT, _B), lambda i: (i, 0)),
        out_shape=jax.ShapeDtypeStruct((_VOCAB, _B), jnp.float32),
    )(W, x, bcol)
    return out_t.T


def kernel(inputs, table, W, b):
    quarter = _D // 4
    sigma = jnp.stack(
        [jnp.arange(quarter), jnp.arange(quarter) + quarter], axis=1
    ).reshape(-1)
    sigma = jnp.concatenate([sigma, sigma + _D // 2])
    table_b = jax.lax.bitcast_convert_type(
        jnp.take(table, sigma, axis=1).astype(jnp.bfloat16).reshape(
            _VOCAB, _D // 2, 2),
        jnp.int32)
    idx2 = inputs.reshape(_NW, _ROWS_PER_W * _HIST)
    x = _sc_pool(idx2, table_b)
    return _project(x, W, b.reshape(_VOCAB, 1))


# speedup vs baseline: 1.4919x; 1.4919x over previous
import functools

import jax
import jax.numpy as jnp
from jax import lax
from jax.experimental import pallas as pl
from jax.experimental.pallas import tpu as pltpu
from jax.experimental.pallas import tpu_sc as plsc

_VOCAB = 100000
_D = 64
_B = 1024
_HIST = 50

_NC, _NS = 2, 16
_NW = _NC * _NS
_ROWS_PER_W = _B // _NW
_GSZ = 8
_NG = _ROWS_PER_W // _GSZ
_IPG = _GSZ * _HIST
_VPG = _IPG // 16
_LANES = 16
_mesh = plsc.VectorSubcoreMesh(core_axis_name="c", subcore_axis_name="s")


@functools.partial(
    pl.kernel,
    out_type=jax.ShapeDtypeStruct((_B, _D), jnp.float32),
    mesh=_mesh,
    scratch_types=[
        pltpu.VMEM((_ROWS_PER_W * _HIST,), jnp.int32),
        pltpu.VMEM((_NG, _IPG, _D // 2), jnp.int32),
        pltpu.VMEM((_ROWS_PER_W, _D), jnp.float32),
        [pltpu.SemaphoreType.DMA for _ in range(_NG)],
    ],
    compiler_params=pltpu.CompilerParams(use_tc_tiling_on_sc=False),
)
def _sc_pool(idx_hbm, table_hbm, x_hbm, idx_v, rows_v, xout_v, sems):
    wid = lax.axis_index("s") * _NC + lax.axis_index("c")
    base = wid * _ROWS_PER_W
    pltpu.sync_copy(idx_hbm.at[wid], idx_v)

    def issue_group(g):
        for u in range(_VPG):
            iv = idx_v[pl.ds(g * _IPG + u * _LANES, _LANES)]
            pltpu.async_copy(table_hbm.at[iv],
                             rows_v.at[g, pl.ds(u * _LANES, _LANES)], sems[g])

    issue_group(0)
    issue_group(1)

    for g in range(_NG):
        if g + 2 < _NG:
            issue_group(g + 2)
        for u in range(_VPG):
            iv = idx_v[pl.ds(g * _IPG + u * _LANES, _LANES)]
            pltpu.make_async_copy(
                table_hbm.at[iv],
                rows_v.at[g, pl.ds(u * _LANES, _LANES)], sems[g]).wait()
        for r in range(_GSZ):
            def acc_body(j, accs):
                out = []
                shift = jnp.full((_LANES,), 16, jnp.int32)
                mask = jnp.full((_LANES,), -65536, jnp.int32)
                for c in range(2):
                    w = rows_v[g, r * _HIST + j, pl.ds(_LANES * c, _LANES)]
                    lo = lax.bitcast_convert_type(
                        lax.shift_left(w, shift), jnp.float32)
                    hi = lax.bitcast_convert_type(
                        lax.bitwise_and(w, mask), jnp.float32)
                    out.append(accs[2 * c] + lo)
                    out.append(accs[2 * c + 1] + hi)
                return tuple(out)

            accs = lax.fori_loop(
                0, _HIST, acc_body,
                tuple(jnp.zeros((_LANES,), jnp.float32) for _ in range(4)))
            for k in range(4):
                xout_v[g * _GSZ + r, pl.ds(k * _LANES, _LANES)] = (
                    accs[k] * (1.0 / _HIST))

    pltpu.sync_copy(xout_v, x_hbm.at[pl.ds(base, _ROWS_PER_W)])


_VT = 4096


def _mm_body(w_ref, x_ref, b_ref, o_ref):
    o_ref[...] = lax.dot_general(
        w_ref[...], x_ref[...],
        dimension_numbers=(((1,), (1,)), ((), ())),
        preferred_element_type=jnp.float32,
    ) + b_ref[...]


def _project(x, W, bcol):
    out_t = pl.pallas_call(
        _mm_body,
        grid=(pl.cdiv(_VOCAB, _VT),),
        in_specs=[
            pl.BlockSpec((_VT, _D), lambda i: (i, 0)),
            pl.BlockSpec((_B, _D), lambda i: (0, 0)),
            pl.BlockSpec((_VT, 1), lambda i: (i, 0)),
        ],
        out_specs=pl.BlockSpec((_VT, _B), lambda i: (i, 0)),
        out_shape=jax.ShapeDtypeStruct((_VOCAB, _B), jnp.float32),
    )(W, x, bcol)
    return out_t.T


def kernel(inputs, table, W, b):
    table_b = jax.lax.bitcast_convert_type(
        table.reshape(_VOCAB, 2, 2, 16).transpose(0, 1, 3, 2)
        .astype(jnp.bfloat16).reshape(_VOCAB, _D // 2, 2),
        jnp.int32)
    idx2 = inputs.reshape(_NW, _ROWS_PER_W * _HIST)
    x = _sc_pool(idx2, table_b)
    return _project(x, W, b.reshape(_VOCAB, 1))


# speedup vs baseline: 1.5107x; 1.0126x over previous
import functools

import jax
import jax.numpy as jnp
from jax import lax
from jax.experimental import pallas as pl
from jax.experimental.pallas import tpu as pltpu
from jax.experimental.pallas import tpu_sc as plsc

_VOCAB = 100000
_D = 64
_B = 1024
_HIST = 50

_NC, _NS = 2, 16
_NW = _NC * _NS
_ROWS_PER_W = _B // _NW
_GSZ = 8
_NG = _ROWS_PER_W // _GSZ
_IPG = _GSZ * _HIST
_VPG = _IPG // 16
_LANES = 16
_mesh = plsc.VectorSubcoreMesh(core_axis_name="c", subcore_axis_name="s")


@functools.partial(
    pl.kernel,
    out_type=jax.ShapeDtypeStruct((_B, _D), jnp.float32),
    mesh=_mesh,
    scratch_types=[
        pltpu.VMEM((_ROWS_PER_W * _HIST,), jnp.int32),
        pltpu.VMEM((_NG, _IPG, _D // 2), jnp.int32),
        pltpu.VMEM((_ROWS_PER_W, _D), jnp.float32),
        [pltpu.SemaphoreType.DMA for _ in range(_NG)],
    ],
    compiler_params=pltpu.CompilerParams(use_tc_tiling_on_sc=False),
)
def _sc_pool(idx_hbm, table_hbm, x_hbm, idx_v, rows_v, xout_v, sems):
    wid = lax.axis_index("s") * _NC + lax.axis_index("c")
    base = wid * _ROWS_PER_W
    pltpu.sync_copy(idx_hbm.at[wid], idx_v)

    def issue_group(g):
        for u in range(_VPG):
            iv = idx_v[pl.ds(g * _IPG + u * _LANES, _LANES)]
            pltpu.async_copy(table_hbm.at[iv],
                             rows_v.at[g, pl.ds(u * _LANES, _LANES)], sems[g])

    issue_group(0)
    issue_group(1)

    for g in range(_NG):
        if g + 2 < _NG:
            issue_group(g + 2)
        for u in range(_VPG):
            iv = idx_v[pl.ds(g * _IPG + u * _LANES, _LANES)]
            pltpu.make_async_copy(
                table_hbm.at[iv],
                rows_v.at[g, pl.ds(u * _LANES, _LANES)], sems[g]).wait()
        for r in range(_GSZ):
            def acc_body(j, accs):
                out = []
                shift = jnp.full((_LANES,), 16, jnp.int32)
                mask = jnp.full((_LANES,), -65536, jnp.int32)
                for c in range(2):
                    w = rows_v[g, r * _HIST + j, pl.ds(_LANES * c, _LANES)]
                    lo = lax.bitcast_convert_type(
                        lax.shift_left(w, shift), jnp.float32)
                    hi = lax.bitcast_convert_type(
                        lax.bitwise_and(w, mask), jnp.float32)
                    out.append(accs[2 * c] + lo)
                    out.append(accs[2 * c + 1] + hi)
                return tuple(out)

            accs = lax.fori_loop(
                0, _HIST, acc_body,
                tuple(jnp.zeros((_LANES,), jnp.float32) for _ in range(4)))
            for k in range(4):
                xout_v[g * _GSZ + r, pl.ds(k * _LANES, _LANES)] = (
                    accs[k] * (1.0 / _HIST))

    pltpu.sync_copy(xout_v, x_hbm.at[pl.ds(base, _ROWS_PER_W)])


_VT = 5120


def _mm_body(w_ref, x_ref, b_ref, o_ref):
    o_ref[...] = lax.dot_general(
        w_ref[...], x_ref[...],
        dimension_numbers=(((1,), (1,)), ((), ())),
        preferred_element_type=jnp.float32,
    ) + b_ref[...]


def _project(x, W, bcol):
    out_t = pl.pallas_call(
        _mm_body,
        grid=(pl.cdiv(_VOCAB, _VT),),
        in_specs=[
            pl.BlockSpec((_VT, _D), lambda i: (i, 0)),
            pl.BlockSpec((_B, _D), lambda i: (0, 0)),
            pl.BlockSpec((_VT, 1), lambda i: (i, 0)),
        ],
        out_specs=pl.BlockSpec((_VT, _B), lambda i: (i, 0)),
        out_shape=jax.ShapeDtypeStruct((_VOCAB, _B), jnp.float32),
    )(W, x, bcol)
    return out_t.T


def kernel(inputs, table, W, b):
    table_b = jax.lax.bitcast_convert_type(
        table.reshape(_VOCAB, 2, 2, 16).transpose(0, 1, 3, 2)
        .astype(jnp.bfloat16).reshape(_VOCAB, _D // 2, 2),
        jnp.int32)
    idx2 = inputs.reshape(_NW, _ROWS_PER_W * _HIST)
    x = _sc_pool(idx2, table_b)
    return _project(x.astype(jnp.bfloat16), W.astype(jnp.bfloat16),
                    b.reshape(_VOCAB, 1))


# speedup vs baseline: 1.5112x; 1.0003x over previous
import functools

import jax
import jax.numpy as jnp
from jax import lax
from jax.experimental import pallas as pl
from jax.experimental.pallas import tpu as pltpu
from jax.experimental.pallas import tpu_sc as plsc

_VOCAB = 100000
_D = 64
_B = 1024
_HIST = 50

_NC, _NS = 2, 16
_NW = _NC * _NS
_ROWS_PER_W = _B // _NW
_GSZ = 8
_NG = _ROWS_PER_W // _GSZ
_IPG = _GSZ * _HIST
_VPG = _IPG // 16
_LANES = 16
_mesh = plsc.VectorSubcoreMesh(core_axis_name="c", subcore_axis_name="s")


@functools.partial(
    pl.kernel,
    out_type=jax.ShapeDtypeStruct((_B, _D), jnp.float32),
    mesh=_mesh,
    scratch_types=[
        pltpu.VMEM((_ROWS_PER_W * _HIST,), jnp.int32),
        pltpu.VMEM((_NG, _IPG, _D // 2), jnp.int32),
        pltpu.VMEM((_ROWS_PER_W, _D), jnp.float32),
        [pltpu.SemaphoreType.DMA for _ in range(_NG)],
    ],
    compiler_params=pltpu.CompilerParams(use_tc_tiling_on_sc=False),
)
def _sc_pool(idx_hbm, table_hbm, x_hbm, idx_v, rows_v, xout_v, sems):
    wid = lax.axis_index("s") * _NC + lax.axis_index("c")
    base = wid * _ROWS_PER_W
    pltpu.sync_copy(idx_hbm.at[wid], idx_v)

    def issue_group(g):
        for u in range(_VPG):
            iv = idx_v[pl.ds(g * _IPG + u * _LANES, _LANES)]
            pltpu.async_copy(table_hbm.at[iv],
                             rows_v.at[g, pl.ds(u * _LANES, _LANES)], sems[g])

    issue_group(0)
    issue_group(1)

    for g in range(_NG):
        if g + 2 < _NG:
            issue_group(g + 2)
        for u in range(_VPG):
            iv = idx_v[pl.ds(g * _IPG + u * _LANES, _LANES)]
            pltpu.make_async_copy(
                table_hbm.at[iv],
                rows_v.at[g, pl.ds(u * _LANES, _LANES)], sems[g]).wait()
        for r in range(_GSZ):
            def acc_body(j, accs):
                out = []
                shift = jnp.full((_LANES,), 16, jnp.int32)
                mask = jnp.full((_LANES,), -65536, jnp.int32)
                for c in range(2):
                    w = rows_v[g, r * _HIST + j, pl.ds(_LANES * c, _LANES)]
                    lo = lax.bitcast_convert_type(
                        lax.shift_left(w, shift), jnp.float32)
                    hi = lax.bitcast_convert_type(
                        lax.bitwise_and(w, mask), jnp.float32)
                    out.append(accs[2 * c] + lo)
                    out.append(accs[2 * c + 1] + hi)
                return tuple(out)

            accs = lax.fori_loop(
                0, _HIST, acc_body,
                tuple(jnp.zeros((_LANES,), jnp.float32) for _ in range(4)))
            for k in range(4):
                xout_v[g * _GSZ + r, pl.ds(k * _LANES, _LANES)] = (
                    accs[k] * (1.0 / _HIST))

    pltpu.sync_copy(xout_v, x_hbm.at[pl.ds(base, _ROWS_PER_W)])


_VT = 5632


def _mm_body(w_ref, x_ref, b_ref, o_ref):
    o_ref[...] = lax.dot_general(
        w_ref[...], x_ref[...],
        dimension_numbers=(((1,), (1,)), ((), ())),
        preferred_element_type=jnp.float32,
    ) + b_ref[...]


def _project(x, W, bcol):
    out_t = pl.pallas_call(
        _mm_body,
        grid=(pl.cdiv(_VOCAB, _VT),),
        in_specs=[
            pl.BlockSpec((_VT, _D), lambda i: (i, 0)),
            pl.BlockSpec((_B, _D), lambda i: (0, 0)),
            pl.BlockSpec((_VT, 1), lambda i: (i, 0)),
        ],
        out_specs=pl.BlockSpec((_VT, _B), lambda i: (i, 0)),
        out_shape=jax.ShapeDtypeStruct((_VOCAB, _B), jnp.float32),
    )(W, x, bcol)
    return out_t.T


def kernel(inputs, table, W, b):
    table_b = jax.lax.bitcast_convert_type(
        table.reshape(_VOCAB, 2, 2, 16).transpose(0, 1, 3, 2)
        .astype(jnp.bfloat16).reshape(_VOCAB, _D // 2, 2),
        jnp.int32)
    idx2 = inputs.reshape(_NW, _ROWS_PER_W * _HIST)
    x = _sc_pool(idx2, table_b)
    return _project(x.astype(jnp.bfloat16), W.astype(jnp.bfloat16),
                    b.reshape(_VOCAB, 1))
